# baseline (speedup 1.0000x reference)
import jax
import jax.numpy as jnp
from jax import lax
from jax.experimental import pallas as pl
from jax.experimental.pallas import tpu as pltpu
from jax.experimental.pallas import tpu_sc as plsc

B = 4096
L = 200
EMBED = 64
HIDDEN = 128
OUT = 2

NC = 2
NS = 16
NW = NC * NS
ROWS_PER_W = B // NW
IDX_PER_W = ROWS_PER_W * L
SPLIT_A = 104
SPLIT_B = L - SPLIT_A
NLG = HIDDEN // 16


def _pool_body(x_hbm, tw_hbm, out_hbm, idx_v, buf0, buf1, buf2, stage,
               sem0, sem1, sem2, semb0, semb1, semb2):
    wid = lax.axis_index("s") * NC + lax.axis_index("c")
    base = wid * ROWS_PER_W

    pltpu.sync_copy(x_hbm.at[pl.ds(base * L, IDX_PER_W)], idx_v)

    bufs = (buf0, buf1, buf2)
    sems = ((sem0, semb0), (sem1, semb1), (sem2, semb2))
    inv_l = jnp.float32(1.0 / L)

    def issue(i, buf, sem):
        off = pl.multiple_of(i * L, 8)
        pltpu.async_copy(
            tw_hbm.at[idx_v.at[pl.ds(off, SPLIT_A)]],
            buf.at[pl.ds(0, SPLIT_A)], sem[0])
        off2 = pl.multiple_of(i * L + SPLIT_A, 8)
        pltpu.async_copy(
            tw_hbm.at[idx_v.at[pl.ds(off2, SPLIT_B)]],
            buf.at[pl.ds(SPLIT_A, SPLIT_B)], sem[1])

    def drain_a(buf, sem):
        pltpu.make_async_copy(tw_hbm.at[pl.ds(0, SPLIT_A)],
                              buf.at[pl.ds(0, SPLIT_A)], sem[0]).wait()

    def drain_b(buf, sem):
        pltpu.make_async_copy(tw_hbm.at[pl.ds(0, SPLIT_B)],
                              buf.at[pl.ds(SPLIT_A, SPLIT_B)], sem[1]).wait()

    def reduce_range(buf, lo, n, accs):
        def red(j, a):
            a = list(a)
            for dj in range(8):
                r = lo + j * 8 + dj
                for c in range(NLG):
                    a[c] = a[c] + buf[r, pl.ds(c * 16, 16)]
            return tuple(a)

        return lax.fori_loop(0, n // 8, red, accs)

    def process(buf, sem, i):
        z = jnp.zeros((16,), jnp.float32)
        drain_a(buf, sem)
        accs = reduce_range(buf, 0, SPLIT_A, (z,) * NLG)
        drain_b(buf, sem)
        accs = reduce_range(buf, SPLIT_A, SPLIT_B, accs)
        for c in range(NLG):
            stage[i, pl.ds(c * 16, 16)] = accs[c] * inv_l

    issue(0, bufs[0], sems[0])
    issue(1, bufs[1], sems[1])

    def outer(i3, carry):
        for k in range(3):
            i = i3 * 3 + k

            @pl.when(i + 2 < ROWS_PER_W)
            def _():
                issue(i + 2, bufs[(k + 2) % 3], sems[(k + 2) % 3])

            process(bufs[k], sems[k], i)
        return carry

    nfull = ROWS_PER_W // 3
    lax.fori_loop(0, nfull, outer, 0)
    for i in range(nfull * 3, ROWS_PER_W):
        k = i % 3
        process(bufs[k], sems[k], i)
    pltpu.sync_copy(stage, out_hbm.at[pl.ds(base, ROWS_PER_W)])


_pool = pl.kernel(
    _pool_body,
    out_type=jax.ShapeDtypeStruct((B, HIDDEN), jnp.float32),
    mesh=plsc.VectorSubcoreMesh(
        core_axis_name="c", subcore_axis_name="s",
        num_cores=NC, num_subcores=NS),
    scratch_types=[
        pltpu.VMEM((IDX_PER_W,), jnp.int32),
        pltpu.VMEM((L, HIDDEN), jnp.float32),
        pltpu.VMEM((L, HIDDEN), jnp.float32),
        pltpu.VMEM((L, HIDDEN), jnp.float32),
        pltpu.VMEM((ROWS_PER_W, HIDDEN), jnp.float32),
        pltpu.SemaphoreType.DMA,
        pltpu.SemaphoreType.DMA,
        pltpu.SemaphoreType.DMA,
        pltpu.SemaphoreType.DMA,
        pltpu.SemaphoreType.DMA,
        pltpu.SemaphoreType.DMA,
    ],
    compiler_params=pltpu.CompilerParams(use_tc_tiling_on_sc=False),
)


def _head_body(p_ref, b1_ref, w2_ref, b2_ref, o_ref):
    h = jnp.maximum(p_ref[...] + b1_ref[...], 0.0)
    o_ref[...] = (
        jnp.dot(h, w2_ref[...], preferred_element_type=jnp.float32)
        + b2_ref[...])


_HEAD_BLK = 512


def _head(pooled, b1, W2, b2):
    return pl.pallas_call(
        _head_body,
        grid=(B // _HEAD_BLK,),
        in_specs=[
            pl.BlockSpec((_HEAD_BLK, HIDDEN), lambda i: (i, 0)),
            pl.BlockSpec((1, HIDDEN), lambda i: (0, 0)),
            pl.BlockSpec((HIDDEN, OUT), lambda i: (0, 0)),
            pl.BlockSpec((1, OUT), lambda i: (0, 0)),
        ],
        out_specs=pl.BlockSpec((_HEAD_BLK, OUT), lambda i: (i, 0)),
        out_shape=jax.ShapeDtypeStruct((B, OUT), jnp.float32),
    )(pooled, b1, W2, b2)


@jax.jit
def kernel(x, table, W1, b1, W2, b2):
    tw = jnp.dot(table, W1, preferred_element_type=jnp.float32)
    x1 = x.reshape(B * L)
    pooled = _pool(x1, tw)
    return _head(pooled, b1.reshape(1, HIDDEN), W2, b2.reshape(1, OUT))


# speedup vs baseline: 1.0105x; 1.0105x over previous
import jax
import jax.numpy as jnp
from jax import lax
from jax.experimental import pallas as pl
from jax.experimental.pallas import tpu as pltpu
from jax.experimental.pallas import tpu_sc as plsc

B = 4096
L = 200
EMBED = 64
HIDDEN = 128
OUT = 2

NC = 2
NS = 16
NW = NC * NS
ROWS_PER_W = B // NW
IDX_PER_W = ROWS_PER_W * L
SPLIT_A = 104
SPLIT_B = L - SPLIT_A
NLG = HIDDEN // 16


def _pool_body(x_hbm, tw_hbm, b1_hbm, w2t_hbm, b2_hbm, out_hbm,
               idx_v, buf0, buf1, buf2, stageo, pb1, pw2, pb2,
               sem0, sem1, sem2, semb0, semb1, semb2):
    wid = lax.axis_index("s") * NC + lax.axis_index("c")
    base = wid * ROWS_PER_W

    pltpu.sync_copy(x_hbm.at[pl.ds(base * L, IDX_PER_W)], idx_v)
    pltpu.sync_copy(b1_hbm, pb1)
    pltpu.sync_copy(w2t_hbm, pw2)
    pltpu.sync_copy(b2_hbm, pb2)

    bufs = (buf0, buf1, buf2)
    sems = ((sem0, semb0), (sem1, semb1), (sem2, semb2))
    inv_l = jnp.float32(1.0 / L)
    lane = lax.iota(jnp.int32, 16)
    b2vec = pb2[pl.ds(0, 16)]

    def issue(i, buf, sem):
        off = pl.multiple_of(i * L, 8)
        pltpu.async_copy(
            tw_hbm.at[idx_v.at[pl.ds(off, SPLIT_A)]],
            buf.at[pl.ds(0, SPLIT_A)], sem[0])
        off2 = pl.multiple_of(i * L + SPLIT_A, 8)
        pltpu.async_copy(
            tw_hbm.at[idx_v.at[pl.ds(off2, SPLIT_B)]],
            buf.at[pl.ds(SPLIT_A, SPLIT_B)], sem[1])

    def drain_a(buf, sem):
        pltpu.make_async_copy(tw_hbm.at[pl.ds(0, SPLIT_A)],
                              buf.at[pl.ds(0, SPLIT_A)], sem[0]).wait()

    def drain_b(buf, sem):
        pltpu.make_async_copy(tw_hbm.at[pl.ds(0, SPLIT_B)],
                              buf.at[pl.ds(SPLIT_A, SPLIT_B)], sem[1]).wait()

    def reduce_range(buf, lo, n, accs):
        def red(j, a):
            a = list(a)
            for dj in range(8):
                r = lo + j * 8 + dj
                for c in range(NLG):
                    a[c] = a[c] + buf[r, pl.ds(c * 16, 16)]
            return tuple(a)

        return lax.fori_loop(0, n // 8, red, accs)

    def process(buf, sem, i):
        z = jnp.zeros((16,), jnp.float32)
        drain_a(buf, sem)
        accs = reduce_range(buf, 0, SPLIT_A, (z,) * NLG)
        drain_b(buf, sem)
        accs = reduce_range(buf, SPLIT_A, SPLIT_B, accs)
        o0 = z
        o1 = z
        for c in range(NLG):
            h = jnp.maximum(accs[c] * inv_l + pb1[pl.ds(c * 16, 16)], 0.0)
            o0 = o0 + h * pw2[0, pl.ds(c * 16, 16)]
            o1 = o1 + h * pw2[1, pl.ds(c * 16, 16)]
        s0 = jnp.sum(o0)
        s1 = jnp.sum(o1)
        v = jnp.where(lane == 0, s0, jnp.where(lane == 1, s1, 0.0)) + b2vec
        stageo[i, pl.ds(0, 16)] = v

    issue(0, bufs[0], sems[0])
    issue(1, bufs[1], sems[1])

    def outer(i3, carry):
        for k in range(3):
            i = i3 * 3 + k

            @pl.when(i + 2 < ROWS_PER_W)
            def _():
                issue(i + 2, bufs[(k + 2) % 3], sems[(k + 2) % 3])

            process(bufs[k], sems[k], i)
        return carry

    nfull = ROWS_PER_W // 3
    lax.fori_loop(0, nfull, outer, 0)
    for i in range(nfull * 3, ROWS_PER_W):
        k = i % 3
        process(bufs[k], sems[k], i)
    pltpu.sync_copy(stageo, out_hbm.at[pl.ds(base, ROWS_PER_W)])


_pool = pl.kernel(
    _pool_body,
    out_type=jax.ShapeDtypeStruct((B, 16), jnp.float32),
    mesh=plsc.VectorSubcoreMesh(
        core_axis_name="c", subcore_axis_name="s",
        num_cores=NC, num_subcores=NS),
    scratch_types=[
        pltpu.VMEM((IDX_PER_W,), jnp.int32),
        pltpu.VMEM((L, HIDDEN), jnp.float32),
        pltpu.VMEM((L, HIDDEN), jnp.float32),
        pltpu.VMEM((L, HIDDEN), jnp.float32),
        pltpu.VMEM((ROWS_PER_W, 16), jnp.float32),
        pltpu.VMEM((HIDDEN,), jnp.float32),
        pltpu.VMEM((OUT, HIDDEN), jnp.float32),
        pltpu.VMEM((16,), jnp.float32),
        pltpu.SemaphoreType.DMA,
        pltpu.SemaphoreType.DMA,
        pltpu.SemaphoreType.DMA,
        pltpu.SemaphoreType.DMA,
        pltpu.SemaphoreType.DMA,
        pltpu.SemaphoreType.DMA,
    ],
    compiler_params=pltpu.CompilerParams(
        use_tc_tiling_on_sc=False, needs_layout_passes=False),
)


@jax.jit
def kernel(x, table, W1, b1, W2, b2):
    tw = jnp.dot(table, W1, preferred_element_type=jnp.float32)
    x1 = x.reshape(B * L)
    b2p = jnp.pad(b2, (0, 16 - OUT))
    out16 = _pool(x1, tw, b1, W2.T, b2p)
    return out16[:, :OUT]
